# Initial kernel scaffold; baseline (speedup 1.0000x reference)
import jax
import jax.numpy as jnp
from jax.experimental import pallas as pl


def kernel(x, edge_index, params):
    raise NotImplementedError("write your pallas kernel here")


# speedup vs baseline: 27.3250x; 27.3250x over previous
import functools
import math

import jax
import jax.numpy as jnp
from jax import lax
from jax.experimental import pallas as pl
from jax.experimental.pallas import tpu as pltpu
from jax.experimental.pallas import tpu_sc as plsc

N = 10000
E = 320000
F = 128
N_PAD = 10240
TRASH = 10200
ROWS_PER_TILE = N_PAD // 16
E_PER_TILE = E // 32
MININT = -(2 ** 31)

@functools.cache
def _mesh():
    return plsc.VectorSubcoreMesh(core_axis_name="c", subcore_axis_name="s",
                                  num_cores=2, num_subcores=16)


def _fill_zero_2d(buf, nrows, width):
    def row(i, carry):
        for j in range(width // 16):
            buf[i, pl.ds(j * 16, 16)] = jnp.zeros((16,), jnp.float32)
        return carry
    lax.fori_loop(0, nrows, row, 0)


def _zero_rows(sh_ref, zbuf, r0, zrows):
    for t in range(ROWS_PER_TILE // zrows):
        pltpu.sync_copy(zbuf, sh_ref.at[pl.ds(r0 + t * zrows, zrows)])


def _scalar_pass_body(src_h, dst_h, keep_h, z_h, degp_h, dstm_h,
                      keepv, srcv, dstv, updv, dstmv, deg_sh):
    c = lax.axis_index("c")
    s = lax.axis_index("s")
    wid = s * 2 + c
    r0 = s * ROWS_PER_TILE
    pltpu.sync_copy(keep_h, keepv)
    pltpu.sync_copy(z_h.at[pl.ds(r0, ROWS_PER_TILE)],
                    deg_sh.at[pl.ds(r0, ROWS_PER_TILE)])
    base = wid * E_PER_TILE
    pltpu.sync_copy(src_h.at[pl.ds(base, E_PER_TILE)], srcv)
    pltpu.sync_copy(dst_h.at[pl.ds(base, E_PER_TILE)], dstv)

    def vec(j, carry):
        s16 = srcv[pl.ds(j * 16, 16)]
        d16 = dstv[pl.ds(j * 16, 16)]
        m = plsc.load_gather(keepv, [s16]) * plsc.load_gather(keepv, [d16])
        updv[pl.ds(j * 16, 16)] = m
        dstmv[pl.ds(j * 16, 16)] = jnp.where(
            m > 0.0, d16, N + (d16 & 127))
        return carry

    lax.fori_loop(0, E_PER_TILE // 16, vec, 0)
    plsc.subcore_barrier()
    pltpu.sync_copy(updv, deg_sh.at[dstmv], add=True)
    pltpu.sync_copy(dstmv, dstm_h.at[pl.ds(base, E_PER_TILE)])
    plsc.subcore_barrier()
    pltpu.sync_copy(deg_sh.at[pl.ds(r0, ROWS_PER_TILE)],
                    degp_h.at[c, pl.ds(r0, ROWS_PER_TILE)])


def _scalar_pass(src, dst, keep, zeros_n):
    fn = pl.kernel(
        _scalar_pass_body,
        out_type=(jax.ShapeDtypeStruct((2, N_PAD), jnp.float32),
                  jax.ShapeDtypeStruct((E,), jnp.int32)),
        mesh=_mesh(),
        compiler_params=pltpu.CompilerParams(needs_layout_passes=False),
        scratch_types=[
            pltpu.VMEM((N_PAD,), jnp.float32),
            pltpu.VMEM((E_PER_TILE,), jnp.int32),
            pltpu.VMEM((E_PER_TILE,), jnp.int32),
            pltpu.VMEM((E_PER_TILE,), jnp.float32),
            pltpu.VMEM((E_PER_TILE,), jnp.int32),
            pltpu.VMEM_SHARED((N_PAD,), jnp.float32),
        ],
    )
    return fn(src, dst, keep, zeros_n)


def _make_score_body(ncols):
    def body(hv_h, src_h, dstm_h, z_h, aggp_h, *refs):
        srcv, dstv = refs[0], refs[1]
        hv = refs[2:2 + ncols]
        updv = refs[2 + ncols:2 + 2 * ncols]
        accs = refs[2 + 2 * ncols:2 + 3 * ncols]
        c = lax.axis_index("c")
        s = lax.axis_index("s")
        wid = s * 2 + c
        r0 = s * ROWS_PER_TILE
        for t in range(ncols):
            pltpu.sync_copy(hv_h.at[pl.ds(t * N_PAD, N_PAD)], hv[t])
            pltpu.sync_copy(z_h.at[pl.ds(r0, ROWS_PER_TILE)],
                            accs[t].at[pl.ds(r0, ROWS_PER_TILE)])
        base = wid * E_PER_TILE
        pltpu.sync_copy(src_h.at[pl.ds(base, E_PER_TILE)], srcv)
        pltpu.sync_copy(dstm_h.at[pl.ds(base, E_PER_TILE)], dstv)

        def vec(j, carry):
            s16 = srcv[pl.ds(j * 16, 16)]
            for t in range(ncols):
                updv[t][pl.ds(j * 16, 16)] = plsc.load_gather(hv[t], [s16])
            return carry

        lax.fori_loop(0, E_PER_TILE // 16, vec, 0)
        plsc.subcore_barrier()
        for t in range(ncols):
            pltpu.sync_copy(updv[t], accs[t].at[dstv], add=True)
        plsc.subcore_barrier()
        for t in range(ncols):
            pltpu.sync_copy(accs[t].at[pl.ds(r0, ROWS_PER_TILE)],
                            aggp_h.at[c, pl.ds(t * N_PAD + r0, ROWS_PER_TILE)])
    return body


def _score_pass(hflat, src, dstm, zeros_n, ncols):
    fn = pl.kernel(
        _make_score_body(ncols),
        out_type=jax.ShapeDtypeStruct((2, ncols * N_PAD), jnp.float32),
        mesh=_mesh(),
        compiler_params=pltpu.CompilerParams(needs_layout_passes=False),
        scratch_types=(
            [pltpu.VMEM((E_PER_TILE,), jnp.int32)] * 2
            + [pltpu.VMEM((N_PAD,), jnp.float32)] * ncols
            + [pltpu.VMEM((E_PER_TILE,), jnp.float32)] * ncols
            + [pltpu.VMEM_SHARED((N_PAD,), jnp.float32)] * ncols
        ),
    )
    return fn(hflat, src, dstm, zeros_n)


EB_F = 200


def _edge_pass_body(h4_h, src4_h, dstm_h, z_h, agg4_h,
                    srcf, dstf, rowsv, feat_sh, sem):
    c = lax.axis_index("c")
    s = lax.axis_index("s")
    r0 = s * ROWS_PER_TILE
    ept = E // 16
    for cc in range(2):
        ch = c + 2 * cc
        pltpu.sync_copy(z_h.at[pl.ds(r0, ROWS_PER_TILE)],
                        feat_sh.at[pl.ds(r0, ROWS_PER_TILE)])
        plsc.subcore_barrier()
        base2 = s * ept

        def fblk(blk, carry):
            off = base2 + blk * EB_F
            pltpu.sync_copy(src4_h.at[pl.ds(ch * E + off, EB_F)], srcf)
            pltpu.sync_copy(dstm_h.at[pl.ds(off, EB_F)], dstf)
            pltpu.async_copy(h4_h.at[srcf], rowsv, sem).wait()
            pltpu.sync_copy(rowsv, feat_sh.at[dstf], add=True)
            return carry

        lax.fori_loop(0, ept // EB_F, fblk, 0)
        plsc.subcore_barrier()
        pltpu.sync_copy(feat_sh.at[pl.ds(r0, ROWS_PER_TILE)],
                        agg4_h.at[pl.ds(ch * N_PAD + r0, ROWS_PER_TILE)])
        plsc.subcore_barrier()


def _edge_pass(h4flat, src4, dstm, zeros_l):
    fn = pl.kernel(
        _edge_pass_body,
        out_type=jax.ShapeDtypeStruct((4 * N_PAD, 128), jnp.float32),
        mesh=_mesh(),
        compiler_params=pltpu.CompilerParams(needs_layout_passes=False),
        scratch_types=[
            pltpu.VMEM((EB_F,), jnp.int32),
            pltpu.VMEM((EB_F,), jnp.int32),
            pltpu.VMEM((EB_F, 128), jnp.float32),
            pltpu.VMEM_SHARED((N_PAD, 128), jnp.float32),
            pltpu.SemaphoreType.DMA,
        ],
    )
    return fn(h4flat, src4, dstm, zeros_l)


BN = 256
NB = N_PAD // BN


def _elu(x):
    return jnp.where(x > 0, x, jnp.exp(jnp.minimum(x, 0.0)) - 1.0)


def _prologue_x(out_prev, score_prev, keep_prev):
    t = jnp.tanh(score_prev)
    return _elu(out_prev * t[:, None]) * keep_prev[:, None]


def _matmul_body(has_skip, has_score, raw_x,
                 x_ref, s_ref, k_ref, w_ref, b_ref, d0_ref, d1_ref,
                 h4_ref, hs_ref, skip_ref):
    if raw_x:
        xb = x_ref[...]
    else:
        xb = _prologue_x(x_ref[...], s_ref[...], k_ref[...])
    rsq = lax.rsqrt(d0_ref[...] + d1_ref[...] + 1.0)
    hall = jnp.dot(xb, w_ref[...], preferred_element_type=jnp.float32) + b_ref[...]
    h4 = (hall[:, :512] * rsq[:, None]).reshape(BN, 4, 128)
    h4_ref[...] = jnp.transpose(h4, (1, 0, 2))
    if has_score:
        hs_ref[...] = hall[:, 512] * rsq
    if has_skip:
        skip_ref[...] = hall[:, 528:784]


def _matmul_layer(x_in, score_in, keep_in, wfull, bfull, d0, d1,
                  raw_x, has_skip, kdim, xwidth):
    outs = [jax.ShapeDtypeStruct((4, N_PAD, 128), jnp.float32),
            jax.ShapeDtypeStruct((N_PAD,), jnp.float32)]
    out_specs = [pl.BlockSpec((4, BN, 128), lambda i: (0, i, 0)),
                 pl.BlockSpec((BN,), lambda i: (i,))]
    if has_skip:
        outs.append(jax.ShapeDtypeStruct((N_PAD, 256), jnp.float32))
        out_specs.append(pl.BlockSpec((BN, 256), lambda i: (i, 0)))

    def wrapped(x_ref, s_ref, k_ref, w_ref, b_ref, d0_ref, d1_ref, *orefs):
        skip_ref = orefs[2] if has_skip else None
        _matmul_body(has_skip, True, raw_x, x_ref, s_ref, k_ref, w_ref,
                     b_ref, d0_ref, d1_ref, orefs[0], orefs[1], skip_ref)

    wwidth = wfull.shape[1]
    in_specs = [
        pl.BlockSpec((BN, xwidth), lambda i: (i, 0)),
        pl.BlockSpec((BN,), lambda i: (i,)),
        pl.BlockSpec((BN,), lambda i: (i,)),
        pl.BlockSpec((kdim, wwidth), lambda i: (0, 0)),
        pl.BlockSpec((wwidth,), lambda i: (0,)),
        pl.BlockSpec((BN,), lambda i: (i,)),
        pl.BlockSpec((BN,), lambda i: (i,)),
    ]
    return pl.pallas_call(
        wrapped, grid=(NB,), in_specs=in_specs,
        out_specs=out_specs, out_shape=outs,
    )(x_in, score_in, keep_in, wfull, bfull, d0, d1)


def _combine_body(has_skip, agg_ref, h4_ref, s0_ref, s1_ref, hs_ref,
                  d0_ref, d1_ref, valid_ref, skip_ref, out_ref, score_ref):
    rsq = lax.rsqrt(d0_ref[...] + d1_ref[...] + 1.0)
    gcn = rsq[None, :, None] * (agg_ref[...] + h4_ref[...])
    m = gcn[0:2]
    g = jax.nn.sigmoid(gcn[2:4])
    out = jnp.transpose(g * m, (1, 0, 2)).reshape(BN, 256)
    if has_skip:
        out = out + skip_ref[...]
    out_ref[...] = out
    sc = (s0_ref[...] + s1_ref[...] + hs_ref[...]) * rsq
    score_ref[...] = jnp.where(valid_ref[...] > 0, sc, -jnp.inf)


def _combine(agg4, h4, aggsp, hs, d0, d1, valid, skip):
    has_skip = skip is not None

    def wrapped(*refs):
        if has_skip:
            (agg_ref, h4_ref, s0r, s1r, hs_ref, d0r, d1r, vr, skr,
             out_ref, score_ref) = refs
        else:
            (agg_ref, h4_ref, s0r, s1r, hs_ref, d0r, d1r, vr,
             out_ref, score_ref) = refs
            skr = None
        _combine_body(has_skip, agg_ref, h4_ref, s0r, s1r, hs_ref,
                      d0r, d1r, vr, skr, out_ref, score_ref)

    in_specs = [
        pl.BlockSpec((4, BN, 128), lambda i: (0, i, 0)),
        pl.BlockSpec((4, BN, 128), lambda i: (0, i, 0)),
        pl.BlockSpec((BN,), lambda i: (i,)),
        pl.BlockSpec((BN,), lambda i: (i,)),
        pl.BlockSpec((BN,), lambda i: (i,)),
        pl.BlockSpec((BN,), lambda i: (i,)),
        pl.BlockSpec((BN,), lambda i: (i,)),
        pl.BlockSpec((BN,), lambda i: (i,)),
    ]
    args = [agg4, h4, aggsp[0], aggsp[1], hs, d0, d1, valid]
    if has_skip:
        in_specs.append(pl.BlockSpec((BN, 256), lambda i: (i, 0)))
        args.append(skip)
    return pl.pallas_call(
        wrapped, grid=(NB,), in_specs=in_specs,
        out_specs=[pl.BlockSpec((BN, 256), lambda i: (i, 0)),
                   pl.BlockSpec((BN,), lambda i: (i,))],
        out_shape=[jax.ShapeDtypeStruct((N_PAD, 256), jnp.float32),
                   jax.ShapeDtypeStruct((N_PAD,), jnp.float32)],
    )(*args)


def _topk_body(kk, score_ref, keep_ref):
    v = lax.bitcast_convert_type(score_ref[...], jnp.int32)
    k = jnp.where(v < 0, ~v, v ^ MININT) ^ MININT

    def step(b, res):
        cand = res | (jnp.int32(1) << (31 - b))
        cnt = jnp.sum((k >= (cand ^ MININT)).astype(jnp.int32))
        return jnp.where(cnt >= kk, cand, res)

    res = lax.fori_loop(0, 32, step, jnp.int32(0))
    keep_ref[...] = (k >= (res ^ MININT)).astype(jnp.float32)


def _topk_mask(score, kk):
    return pl.pallas_call(
        functools.partial(_topk_body, kk),
        out_shape=jax.ShapeDtypeStruct((N_PAD,), jnp.float32),
    )(score)


def _final_body(aggp_ref, h3_ref, d0_ref, d1_ref, keep_ref,
                w1_ref, b1_ref, w2_ref, b2_ref, out_ref):
    rsq = lax.rsqrt(d0_ref[...] + d1_ref[...] + 1.0)

    def _col(t):
        sl = pl.ds(t * N_PAD, N_PAD)
        return rsq * (aggp_ref[0, sl] + aggp_ref[1, sl] + h3_ref[sl])

    x3 = 0.5 * (jax.nn.sigmoid(_col(2)) * _col(0)
                + jax.nn.sigmoid(_col(3)) * _col(1))
    s = jnp.where(keep_ref[...] > 0, x3, -jnp.inf).reshape(80, 128)
    ri = (lax.broadcasted_iota(jnp.int32, (80, 128), 0) * 128
          + lax.broadcasted_iota(jnp.int32, (80, 128), 1))
    i32v = lax.broadcasted_iota(jnp.int32, (1, 32), 1)

    def step(i, carry):
        sv, pooled = carry
        mx = jnp.max(sv)
        jj = jnp.min(jnp.where(sv == mx, ri, jnp.int32(2 ** 30)))
        pooled = jnp.where(i32v == i, mx, pooled)
        sv = jnp.where(ri == jj, -jnp.inf, sv)
        return sv, pooled

    _, pooled = lax.fori_loop(0, 30, step,
                              (s, jnp.zeros((1, 32), jnp.float32)))
    t1 = _elu(jnp.dot(pooled, w1_ref[...],
                      preferred_element_type=jnp.float32) + b1_ref[...])
    t2 = jnp.dot(t1, w2_ref[...],
                 preferred_element_type=jnp.float32) + b2_ref[...]
    out_ref[...] = t2


def _final(aggp, h3, d0, d1, keep, w1, b1, w2, b2):
    return pl.pallas_call(
        _final_body,
        out_shape=jax.ShapeDtypeStruct((1, 10), jnp.float32),
    )(aggp, h3, d0, d1, keep, w1, b1, w2, b2)


def _matmul3_body(x_ref, s_ref, k_ref, w_ref, b_ref, d0_ref, d1_ref, *h3_ref):
    xb = _prologue_x(x_ref[...], s_ref[...], k_ref[...])
    rsq = lax.rsqrt(d0_ref[...] + d1_ref[...] + 1.0)
    hall = jnp.dot(xb, w_ref[...], preferred_element_type=jnp.float32) + b_ref[...]
    hs = hall * rsq[:, None]
    for t in range(4):
        h3_ref[t][...] = hs[:, t]


def _matmul3(out2, score2, keep2, w3cat, b3cat, d0, d1):
    return pl.pallas_call(
        _matmul3_body, grid=(NB,),
        in_specs=[
            pl.BlockSpec((BN, 256), lambda i: (i, 0)),
            pl.BlockSpec((BN,), lambda i: (i,)),
            pl.BlockSpec((BN,), lambda i: (i,)),
            pl.BlockSpec((256, 16), lambda i: (0, 0)),
            pl.BlockSpec((16,), lambda i: (0,)),
            pl.BlockSpec((BN,), lambda i: (i,)),
            pl.BlockSpec((BN,), lambda i: (i,)),
        ],
        out_specs=[pl.BlockSpec((BN,), lambda i: (i,))] * 4,
        out_shape=[jax.ShapeDtypeStruct((N_PAD,), jnp.float32)] * 4,
    )(out2, score2, keep2, w3cat, b3cat, d0, d1)


def _concat_weights(p, pre, has_skip):
    ws = [p[f'{pre}_W0'], p[f'{pre}_W1'], p[f'{pre}_Wg0'], p[f'{pre}_Wg1']]
    bs = [p[f'{pre}_b0'], p[f'{pre}_b1'], p[f'{pre}_bg0'], p[f'{pre}_bg1']]
    cols = [jnp.concatenate(ws, axis=1),
            jnp.pad(p[f'{pre}_Ws'], ((0, 0), (0, 15)))]
    bcols = [jnp.concatenate(bs), jnp.pad(p[f'{pre}_bs'], (0, 15))]
    if has_skip:
        cols.append(p[f'{pre}_Wskip'])
        bcols.append(jnp.zeros((256,), jnp.float32))
    return jnp.concatenate(cols, axis=1), jnp.concatenate(bcols)


def kernel(x, edge_index, params):
    p = params
    src = edge_index[0].astype(jnp.int32)
    dst = edge_index[1].astype(jnp.int32)
    xp = jnp.pad(x, ((0, N_PAD - N), (0, 0)))
    valid0 = (jnp.arange(N_PAD) < N).astype(jnp.float32)
    zeros_n = jnp.zeros((N_PAD,), jnp.float32)

    w1full, b1full = _concat_weights(p, 'c1', False)
    w2full, b2full = _concat_weights(p, 'c2', True)
    w3cat = jnp.pad(jnp.concatenate(
        [p['c3_W0'], p['c3_W1'], p['c3_Wg0'], p['c3_Wg1']], axis=1),
        ((0, 0), (0, 12)))
    b3cat = jnp.pad(jnp.concatenate(
        [p['c3_b0'], p['c3_b1'], p['c3_bg0'], p['c3_bg1']]), (0, 12))

    zeros_l = jnp.zeros((N_PAD, 128), jnp.float32)
    src4 = jnp.concatenate([src + ch * N_PAD for ch in range(4)])

    degp1, dstm1 = _scalar_pass(src, dst, valid0, zeros_n)
    d0, d1 = degp1[0], degp1[1]
    h4_1, hs_1 = _matmul_layer(xp, zeros_n, zeros_n, w1full, b1full,
                               d0, d1, True, False, F, F)
    agg4_1 = _edge_pass(h4_1.reshape(4 * N_PAD, 128), src4, dstm1, zeros_l)
    aggsp_1 = _score_pass(hs_1, src, dstm1, zeros_n, 1)
    out1, score1 = _combine(agg4_1.reshape(4, N_PAD, 128), h4_1, aggsp_1,
                            hs_1, d0, d1, valid0, None)
    keep1 = _topk_mask(score1, int(math.ceil(0.5 * N)))

    degp2, dstm2 = _scalar_pass(src, dst, keep1, zeros_n)
    d0, d1 = degp2[0], degp2[1]
    h4_2, hs_2, skip2 = _matmul_layer(out1, score1, keep1, w2full, b2full,
                                      d0, d1, False, True, 256, 256)
    agg4_2 = _edge_pass(h4_2.reshape(4 * N_PAD, 128), src4, dstm2, zeros_l)
    aggsp_2 = _score_pass(hs_2, src, dstm2, zeros_n, 1)
    out2, score2 = _combine(agg4_2.reshape(4, N_PAD, 128), h4_2, aggsp_2,
                            hs_2, d0, d1, keep1, skip2)
    keep2 = _topk_mask(score2, int(math.ceil(0.5 * math.ceil(0.5 * N))))

    degp3, dstm3 = _scalar_pass(src, dst, keep2, zeros_n)
    d0, d1 = degp3[0], degp3[1]
    h3cols = _matmul3(out2, score2, keep2, w3cat, b3cat, d0, d1)
    h3flat = jnp.concatenate(h3cols)
    aggp3 = _score_pass(h3flat, src, dstm3, zeros_n, 4)
    w1p = jnp.pad(p['cls_W1'], ((0, 2), (0, 0)))
    return _final(aggp3, h3flat, d0, d1, keep2,
                  w1p, p['cls_b1'].reshape(1, -1),
                  p['cls_W2'], p['cls_b2'].reshape(1, -1))


# speedup vs baseline: 35.3691x; 1.2944x over previous
import functools
import math

import jax
import jax.numpy as jnp
from jax import lax
from jax.experimental import pallas as pl
from jax.experimental.pallas import tpu as pltpu
from jax.experimental.pallas import tpu_sc as plsc

N = 10000
E = 320000
F = 128
N_PAD = 10240
TRASH = 10200
ROWS_PER_TILE = N_PAD // 16
E_PER_TILE = E // 32
MININT = -(2 ** 31)

@functools.cache
def _mesh():
    return plsc.VectorSubcoreMesh(core_axis_name="c", subcore_axis_name="s",
                                  num_cores=2, num_subcores=16)


def _fill_zero_2d(buf, nrows, width):
    def row(i, carry):
        for j in range(width // 16):
            buf[i, pl.ds(j * 16, 16)] = jnp.zeros((16,), jnp.float32)
        return carry
    lax.fori_loop(0, nrows, row, 0)


def _zero_rows(sh_ref, zbuf, r0, zrows):
    for t in range(ROWS_PER_TILE // zrows):
        pltpu.sync_copy(zbuf, sh_ref.at[pl.ds(r0 + t * zrows, zrows)])


def _scalar_pass_body(src_h, dst_h, keep_h, z_h, degp_h, dstm_h,
                      keepv, srcv, dstv, updv, dstmv, deg_sh):
    c = lax.axis_index("c")
    s = lax.axis_index("s")
    wid = s * 2 + c
    r0 = s * ROWS_PER_TILE
    pltpu.sync_copy(keep_h, keepv)
    pltpu.sync_copy(z_h.at[pl.ds(r0, ROWS_PER_TILE)],
                    deg_sh.at[pl.ds(r0, ROWS_PER_TILE)])
    base = wid * E_PER_TILE
    pltpu.sync_copy(src_h.at[pl.ds(base, E_PER_TILE)], srcv)
    pltpu.sync_copy(dst_h.at[pl.ds(base, E_PER_TILE)], dstv)

    def vec(j, carry):
        s16 = srcv[pl.ds(j * 16, 16)]
        d16 = dstv[pl.ds(j * 16, 16)]
        m = plsc.load_gather(keepv, [s16]) * plsc.load_gather(keepv, [d16])
        updv[pl.ds(j * 16, 16)] = m
        dstmv[pl.ds(j * 16, 16)] = jnp.where(
            m > 0.0, d16, N + (d16 & 127))
        return carry

    lax.fori_loop(0, E_PER_TILE // 16, vec, 0)
    plsc.subcore_barrier()
    pltpu.sync_copy(updv, deg_sh.at[dstmv], add=True)
    pltpu.sync_copy(dstmv, dstm_h.at[pl.ds(base, E_PER_TILE)])
    plsc.subcore_barrier()
    pltpu.sync_copy(deg_sh.at[pl.ds(r0, ROWS_PER_TILE)],
                    degp_h.at[c, pl.ds(r0, ROWS_PER_TILE)])


def _scalar_pass(src, dst, keep, zeros_n):
    fn = pl.kernel(
        _scalar_pass_body,
        out_type=(jax.ShapeDtypeStruct((2, N_PAD), jnp.float32),
                  jax.ShapeDtypeStruct((E,), jnp.int32)),
        mesh=_mesh(),
        compiler_params=pltpu.CompilerParams(needs_layout_passes=False),
        scratch_types=[
            pltpu.VMEM((N_PAD,), jnp.float32),
            pltpu.VMEM((E_PER_TILE,), jnp.int32),
            pltpu.VMEM((E_PER_TILE,), jnp.int32),
            pltpu.VMEM((E_PER_TILE,), jnp.float32),
            pltpu.VMEM((E_PER_TILE,), jnp.int32),
            pltpu.VMEM_SHARED((N_PAD,), jnp.float32),
        ],
    )
    return fn(src, dst, keep, zeros_n)


def _make_score_body(ncols):
    def body(hv_h, src_h, dstm_h, z_h, aggp_h, *refs):
        srcv, dstv = refs[0], refs[1]
        hv = refs[2:2 + ncols]
        updv = refs[2 + ncols:2 + 2 * ncols]
        accs = refs[2 + 2 * ncols:2 + 3 * ncols]
        c = lax.axis_index("c")
        s = lax.axis_index("s")
        wid = s * 2 + c
        r0 = s * ROWS_PER_TILE
        for t in range(ncols):
            pltpu.sync_copy(hv_h.at[pl.ds(t * N_PAD, N_PAD)], hv[t])
            pltpu.sync_copy(z_h.at[pl.ds(r0, ROWS_PER_TILE)],
                            accs[t].at[pl.ds(r0, ROWS_PER_TILE)])
        base = wid * E_PER_TILE
        pltpu.sync_copy(src_h.at[pl.ds(base, E_PER_TILE)], srcv)
        pltpu.sync_copy(dstm_h.at[pl.ds(base, E_PER_TILE)], dstv)

        def vec(j, carry):
            s16 = srcv[pl.ds(j * 16, 16)]
            for t in range(ncols):
                updv[t][pl.ds(j * 16, 16)] = plsc.load_gather(hv[t], [s16])
            return carry

        lax.fori_loop(0, E_PER_TILE // 16, vec, 0)
        plsc.subcore_barrier()
        for t in range(ncols):
            pltpu.sync_copy(updv[t], accs[t].at[dstv], add=True)
        plsc.subcore_barrier()
        for t in range(ncols):
            pltpu.sync_copy(accs[t].at[pl.ds(r0, ROWS_PER_TILE)],
                            aggp_h.at[c, pl.ds(t * N_PAD + r0, ROWS_PER_TILE)])
    return body


def _score_pass(hflat, src, dstm, zeros_n, ncols):
    fn = pl.kernel(
        _make_score_body(ncols),
        out_type=jax.ShapeDtypeStruct((2, ncols * N_PAD), jnp.float32),
        mesh=_mesh(),
        compiler_params=pltpu.CompilerParams(needs_layout_passes=False),
        scratch_types=(
            [pltpu.VMEM((E_PER_TILE,), jnp.int32)] * 2
            + [pltpu.VMEM((N_PAD,), jnp.float32)] * ncols
            + [pltpu.VMEM((E_PER_TILE,), jnp.float32)] * ncols
            + [pltpu.VMEM_SHARED((N_PAD,), jnp.float32)] * ncols
        ),
    )
    return fn(hflat, src, dstm, zeros_n)


EB_F = 160


def _edge_pass_body(h4_h, src4_h, dstm_h, z_h, agg4_h,
                    srcb0, srcb1, dstb, rows0, rows1, feat_sh, sem0, sem1):
    c = lax.axis_index("c")
    s = lax.axis_index("s")
    r0 = s * ROWS_PER_TILE
    ept = E // 16
    nblk = ept // EB_F
    for cc in range(2):
        ch = c + 2 * cc
        pltpu.sync_copy(z_h.at[pl.ds(r0, ROWS_PER_TILE)],
                        feat_sh.at[pl.ds(r0, ROWS_PER_TILE)])
        plsc.subcore_barrier()
        base2 = s * ept

        def gstart(i, srcb, rows, sem):
            pltpu.sync_copy(src4_h.at[pl.ds(ch * E + base2 + i * EB_F, EB_F)],
                            srcb)
            pltpu.make_async_copy(h4_h.at[srcb], rows, sem).start()

        def gwait(srcb, rows, sem):
            pltpu.make_async_copy(h4_h.at[srcb], rows, sem).wait()

        def scat(i, rows):
            pltpu.sync_copy(dstm_h.at[pl.ds(base2 + i * EB_F, EB_F)], dstb)
            pltpu.sync_copy(rows, feat_sh.at[dstb], add=True)

        gstart(0, srcb0, rows0, sem0)

        def pair(p, carry):
            i0 = 2 * p
            gwait(srcb0, rows0, sem0)
            gstart(i0 + 1, srcb1, rows1, sem1)
            scat(i0, rows0)
            gwait(srcb1, rows1, sem1)
            gstart(i0 + 2, srcb0, rows0, sem0)
            scat(i0 + 1, rows1)
            return carry

        lax.fori_loop(0, (nblk - 1) // 2, pair, 0)
        gwait(srcb0, rows0, sem0)
        scat(nblk - 1, rows0)
        plsc.subcore_barrier()
        pltpu.sync_copy(feat_sh.at[pl.ds(r0, ROWS_PER_TILE)],
                        agg4_h.at[pl.ds(ch * N_PAD + r0, ROWS_PER_TILE)])
        plsc.subcore_barrier()


def _edge_pass(h4flat, src4, dstm, zeros_l):
    fn = pl.kernel(
        _edge_pass_body,
        out_type=jax.ShapeDtypeStruct((4 * N_PAD, 128), jnp.float32),
        mesh=_mesh(),
        compiler_params=pltpu.CompilerParams(needs_layout_passes=False),
        scratch_types=[
            pltpu.VMEM((EB_F,), jnp.int32),
            pltpu.VMEM((EB_F,), jnp.int32),
            pltpu.VMEM((EB_F,), jnp.int32),
            pltpu.VMEM((EB_F, 128), jnp.float32),
            pltpu.VMEM((EB_F, 128), jnp.float32),
            pltpu.VMEM_SHARED((N_PAD, 128), jnp.float32),
            pltpu.SemaphoreType.DMA,
            pltpu.SemaphoreType.DMA,
        ],
    )
    return fn(h4flat, src4, dstm, zeros_l)


BN = 256
NB = N_PAD // BN


def _elu(x):
    return jnp.where(x > 0, x, jnp.exp(jnp.minimum(x, 0.0)) - 1.0)


def _prologue_x(out_prev, score_prev, keep_prev):
    t = jnp.tanh(score_prev)
    return _elu(out_prev * t[:, None]) * keep_prev[:, None]


def _matmul_body(has_skip, has_score, raw_x,
                 x_ref, s_ref, k_ref, w_ref, b_ref, d0_ref, d1_ref,
                 h4_ref, hs_ref, skip_ref):
    if raw_x:
        xb = x_ref[...]
    else:
        xb = _prologue_x(x_ref[...], s_ref[...], k_ref[...])
    rsq = lax.rsqrt(d0_ref[...] + d1_ref[...] + 1.0)
    hall = jnp.dot(xb, w_ref[...], preferred_element_type=jnp.float32) + b_ref[...]
    h4 = (hall[:, :512] * rsq[:, None]).reshape(BN, 4, 128)
    h4_ref[...] = jnp.transpose(h4, (1, 0, 2))
    if has_score:
        hs_ref[...] = hall[:, 512] * rsq
    if has_skip:
        skip_ref[...] = hall[:, 528:784]


def _matmul_layer(x_in, score_in, keep_in, wfull, bfull, d0, d1,
                  raw_x, has_skip, kdim, xwidth):
    outs = [jax.ShapeDtypeStruct((4, N_PAD, 128), jnp.float32),
            jax.ShapeDtypeStruct((N_PAD,), jnp.float32)]
    out_specs = [pl.BlockSpec((4, BN, 128), lambda i: (0, i, 0)),
                 pl.BlockSpec((BN,), lambda i: (i,))]
    if has_skip:
        outs.append(jax.ShapeDtypeStruct((N_PAD, 256), jnp.float32))
        out_specs.append(pl.BlockSpec((BN, 256), lambda i: (i, 0)))

    def wrapped(x_ref, s_ref, k_ref, w_ref, b_ref, d0_ref, d1_ref, *orefs):
        skip_ref = orefs[2] if has_skip else None
        _matmul_body(has_skip, True, raw_x, x_ref, s_ref, k_ref, w_ref,
                     b_ref, d0_ref, d1_ref, orefs[0], orefs[1], skip_ref)

    wwidth = wfull.shape[1]
    in_specs = [
        pl.BlockSpec((BN, xwidth), lambda i: (i, 0)),
        pl.BlockSpec((BN,), lambda i: (i,)),
        pl.BlockSpec((BN,), lambda i: (i,)),
        pl.BlockSpec((kdim, wwidth), lambda i: (0, 0)),
        pl.BlockSpec((wwidth,), lambda i: (0,)),
        pl.BlockSpec((BN,), lambda i: (i,)),
        pl.BlockSpec((BN,), lambda i: (i,)),
    ]
    return pl.pallas_call(
        wrapped, grid=(NB,), in_specs=in_specs,
        out_specs=out_specs, out_shape=outs,
    )(x_in, score_in, keep_in, wfull, bfull, d0, d1)


def _combine_body(has_skip, agg_ref, h4_ref, s0_ref, s1_ref, hs_ref,
                  d0_ref, d1_ref, valid_ref, skip_ref, out_ref, score_ref):
    rsq = lax.rsqrt(d0_ref[...] + d1_ref[...] + 1.0)
    gcn = rsq[None, :, None] * (agg_ref[...] + h4_ref[...])
    m = gcn[0:2]
    g = jax.nn.sigmoid(gcn[2:4])
    out = jnp.transpose(g * m, (1, 0, 2)).reshape(BN, 256)
    if has_skip:
        out = out + skip_ref[...]
    out_ref[...] = out
    sc = (s0_ref[...] + s1_ref[...] + hs_ref[...]) * rsq
    score_ref[...] = jnp.where(valid_ref[...] > 0, sc, -jnp.inf)


def _combine(agg4, h4, aggsp, hs, d0, d1, valid, skip):
    has_skip = skip is not None

    def wrapped(*refs):
        if has_skip:
            (agg_ref, h4_ref, s0r, s1r, hs_ref, d0r, d1r, vr, skr,
             out_ref, score_ref) = refs
        else:
            (agg_ref, h4_ref, s0r, s1r, hs_ref, d0r, d1r, vr,
             out_ref, score_ref) = refs
            skr = None
        _combine_body(has_skip, agg_ref, h4_ref, s0r, s1r, hs_ref,
                      d0r, d1r, vr, skr, out_ref, score_ref)

    in_specs = [
        pl.BlockSpec((4, BN, 128), lambda i: (0, i, 0)),
        pl.BlockSpec((4, BN, 128), lambda i: (0, i, 0)),
        pl.BlockSpec((BN,), lambda i: (i,)),
        pl.BlockSpec((BN,), lambda i: (i,)),
        pl.BlockSpec((BN,), lambda i: (i,)),
        pl.BlockSpec((BN,), lambda i: (i,)),
        pl.BlockSpec((BN,), lambda i: (i,)),
        pl.BlockSpec((BN,), lambda i: (i,)),
    ]
    args = [agg4, h4, aggsp[0], aggsp[1], hs, d0, d1, valid]
    if has_skip:
        in_specs.append(pl.BlockSpec((BN, 256), lambda i: (i, 0)))
        args.append(skip)
    return pl.pallas_call(
        wrapped, grid=(NB,), in_specs=in_specs,
        out_specs=[pl.BlockSpec((BN, 256), lambda i: (i, 0)),
                   pl.BlockSpec((BN,), lambda i: (i,))],
        out_shape=[jax.ShapeDtypeStruct((N_PAD, 256), jnp.float32),
                   jax.ShapeDtypeStruct((N_PAD,), jnp.float32)],
    )(*args)


def _topk_body(kk, score_ref, keep_ref):
    v = lax.bitcast_convert_type(score_ref[...], jnp.int32)
    k = jnp.where(v < 0, ~v, v ^ MININT) ^ MININT

    def step(b, res):
        cand = res | (jnp.int32(1) << (31 - b))
        cnt = jnp.sum((k >= (cand ^ MININT)).astype(jnp.int32))
        return jnp.where(cnt >= kk, cand, res)

    res = lax.fori_loop(0, 32, step, jnp.int32(0))
    keep_ref[...] = (k >= (res ^ MININT)).astype(jnp.float32)


def _topk_mask(score, kk):
    return pl.pallas_call(
        functools.partial(_topk_body, kk),
        out_shape=jax.ShapeDtypeStruct((N_PAD,), jnp.float32),
    )(score)


def _final_body(aggp_ref, h3_ref, d0_ref, d1_ref, keep_ref,
                w1_ref, b1_ref, w2_ref, b2_ref, out_ref):
    rsq = lax.rsqrt(d0_ref[...] + d1_ref[...] + 1.0)

    def _col(t):
        sl = pl.ds(t * N_PAD, N_PAD)
        return rsq * (aggp_ref[0, sl] + aggp_ref[1, sl] + h3_ref[sl])

    x3 = 0.5 * (jax.nn.sigmoid(_col(2)) * _col(0)
                + jax.nn.sigmoid(_col(3)) * _col(1))
    s = jnp.where(keep_ref[...] > 0, x3, -jnp.inf).reshape(80, 128)
    ri = (lax.broadcasted_iota(jnp.int32, (80, 128), 0) * 128
          + lax.broadcasted_iota(jnp.int32, (80, 128), 1))
    i32v = lax.broadcasted_iota(jnp.int32, (1, 32), 1)

    def step(i, carry):
        sv, pooled = carry
        mx = jnp.max(sv)
        jj = jnp.min(jnp.where(sv == mx, ri, jnp.int32(2 ** 30)))
        pooled = jnp.where(i32v == i, mx, pooled)
        sv = jnp.where(ri == jj, -jnp.inf, sv)
        return sv, pooled

    _, pooled = lax.fori_loop(0, 30, step,
                              (s, jnp.zeros((1, 32), jnp.float32)))
    t1 = _elu(jnp.dot(pooled, w1_ref[...],
                      preferred_element_type=jnp.float32) + b1_ref[...])
    t2 = jnp.dot(t1, w2_ref[...],
                 preferred_element_type=jnp.float32) + b2_ref[...]
    out_ref[...] = t2


def _final(aggp, h3, d0, d1, keep, w1, b1, w2, b2):
    return pl.pallas_call(
        _final_body,
        out_shape=jax.ShapeDtypeStruct((1, 10), jnp.float32),
    )(aggp, h3, d0, d1, keep, w1, b1, w2, b2)


def _matmul3_body(x_ref, s_ref, k_ref, w_ref, b_ref, d0_ref, d1_ref, *h3_ref):
    xb = _prologue_x(x_ref[...], s_ref[...], k_ref[...])
    rsq = lax.rsqrt(d0_ref[...] + d1_ref[...] + 1.0)
    hall = jnp.dot(xb, w_ref[...], preferred_element_type=jnp.float32) + b_ref[...]
    hs = hall * rsq[:, None]
    for t in range(4):
        h3_ref[t][...] = hs[:, t]


def _matmul3(out2, score2, keep2, w3cat, b3cat, d0, d1):
    return pl.pallas_call(
        _matmul3_body, grid=(NB,),
        in_specs=[
            pl.BlockSpec((BN, 256), lambda i: (i, 0)),
            pl.BlockSpec((BN,), lambda i: (i,)),
            pl.BlockSpec((BN,), lambda i: (i,)),
            pl.BlockSpec((256, 16), lambda i: (0, 0)),
            pl.BlockSpec((16,), lambda i: (0,)),
            pl.BlockSpec((BN,), lambda i: (i,)),
            pl.BlockSpec((BN,), lambda i: (i,)),
        ],
        out_specs=[pl.BlockSpec((BN,), lambda i: (i,))] * 4,
        out_shape=[jax.ShapeDtypeStruct((N_PAD,), jnp.float32)] * 4,
    )(out2, score2, keep2, w3cat, b3cat, d0, d1)


def _concat_weights(p, pre, has_skip):
    ws = [p[f'{pre}_W0'], p[f'{pre}_W1'], p[f'{pre}_Wg0'], p[f'{pre}_Wg1']]
    bs = [p[f'{pre}_b0'], p[f'{pre}_b1'], p[f'{pre}_bg0'], p[f'{pre}_bg1']]
    cols = [jnp.concatenate(ws, axis=1),
            jnp.pad(p[f'{pre}_Ws'], ((0, 0), (0, 15)))]
    bcols = [jnp.concatenate(bs), jnp.pad(p[f'{pre}_bs'], (0, 15))]
    if has_skip:
        cols.append(p[f'{pre}_Wskip'])
        bcols.append(jnp.zeros((256,), jnp.float32))
    return jnp.concatenate(cols, axis=1), jnp.concatenate(bcols)


def kernel(x, edge_index, params):
    p = params
    src = edge_index[0].astype(jnp.int32)
    dst = edge_index[1].astype(jnp.int32)
    xp = jnp.pad(x, ((0, N_PAD - N), (0, 0)))
    valid0 = (jnp.arange(N_PAD) < N).astype(jnp.float32)
    zeros_n = jnp.zeros((N_PAD,), jnp.float32)

    w1full, b1full = _concat_weights(p, 'c1', False)
    w2full, b2full = _concat_weights(p, 'c2', True)
    w3cat = jnp.pad(jnp.concatenate(
        [p['c3_W0'], p['c3_W1'], p['c3_Wg0'], p['c3_Wg1']], axis=1),
        ((0, 0), (0, 12)))
    b3cat = jnp.pad(jnp.concatenate(
        [p['c3_b0'], p['c3_b1'], p['c3_bg0'], p['c3_bg1']]), (0, 12))

    zeros_l = jnp.zeros((N_PAD, 128), jnp.float32)
    src4 = jnp.concatenate([src + ch * N_PAD for ch in range(4)])

    degp1, dstm1 = _scalar_pass(src, dst, valid0, zeros_n)
    d0, d1 = degp1[0], degp1[1]
    h4_1, hs_1 = _matmul_layer(xp, zeros_n, zeros_n, w1full, b1full,
                               d0, d1, True, False, F, F)
    agg4_1 = _edge_pass(h4_1.reshape(4 * N_PAD, 128), src4, dstm1, zeros_l)
    aggsp_1 = _score_pass(hs_1, src, dstm1, zeros_n, 1)
    out1, score1 = _combine(agg4_1.reshape(4, N_PAD, 128), h4_1, aggsp_1,
                            hs_1, d0, d1, valid0, None)
    keep1 = _topk_mask(score1, int(math.ceil(0.5 * N)))

    degp2, dstm2 = _scalar_pass(src, dst, keep1, zeros_n)
    d0, d1 = degp2[0], degp2[1]
    h4_2, hs_2, skip2 = _matmul_layer(out1, score1, keep1, w2full, b2full,
                                      d0, d1, False, True, 256, 256)
    agg4_2 = _edge_pass(h4_2.reshape(4 * N_PAD, 128), src4, dstm2, zeros_l)
    aggsp_2 = _score_pass(hs_2, src, dstm2, zeros_n, 1)
    out2, score2 = _combine(agg4_2.reshape(4, N_PAD, 128), h4_2, aggsp_2,
                            hs_2, d0, d1, keep1, skip2)
    keep2 = _topk_mask(score2, int(math.ceil(0.5 * math.ceil(0.5 * N))))

    degp3, dstm3 = _scalar_pass(src, dst, keep2, zeros_n)
    d0, d1 = degp3[0], degp3[1]
    h3cols = _matmul3(out2, score2, keep2, w3cat, b3cat, d0, d1)
    h3flat = jnp.concatenate(h3cols)
    aggp3 = _score_pass(h3flat, src, dstm3, zeros_n, 4)
    w1p = jnp.pad(p['cls_W1'], ((0, 2), (0, 0)))
    return _final(aggp3, h3flat, d0, d1, keep2,
                  w1p, p['cls_b1'].reshape(1, -1),
                  p['cls_W2'], p['cls_b2'].reshape(1, -1))


# speedup vs baseline: 40.7039x; 1.1508x over previous
import functools
import math

import jax
import jax.numpy as jnp
from jax import lax
from jax.experimental import pallas as pl
from jax.experimental.pallas import tpu as pltpu
from jax.experimental.pallas import tpu_sc as plsc

N = 10000
E = 320000
F = 128
N_PAD = 10240
TRASH = 10200
ROWS_PER_TILE = N_PAD // 16
E_PER_TILE = E // 32
MININT = -(2 ** 31)

@functools.cache
def _mesh():
    return plsc.VectorSubcoreMesh(core_axis_name="c", subcore_axis_name="s",
                                  num_cores=2, num_subcores=16)


def _fill_zero_2d(buf, nrows, width):
    def row(i, carry):
        for j in range(width // 16):
            buf[i, pl.ds(j * 16, 16)] = jnp.zeros((16,), jnp.float32)
        return carry
    lax.fori_loop(0, nrows, row, 0)


def _zero_rows(sh_ref, zbuf, r0, zrows):
    for t in range(ROWS_PER_TILE // zrows):
        pltpu.sync_copy(zbuf, sh_ref.at[pl.ds(r0 + t * zrows, zrows)])


def _scalar_pass_body(src_h, dst_h, keep_h, z_h, degp_h, dstm_h,
                      keepv, srcv, dstv, updv, dstmv, deg_sh):
    c = lax.axis_index("c")
    s = lax.axis_index("s")
    wid = s * 2 + c
    r0 = s * ROWS_PER_TILE
    pltpu.sync_copy(keep_h, keepv)
    pltpu.sync_copy(z_h.at[pl.ds(r0, ROWS_PER_TILE)],
                    deg_sh.at[pl.ds(r0, ROWS_PER_TILE)])
    base = wid * E_PER_TILE
    pltpu.sync_copy(src_h.at[pl.ds(base, E_PER_TILE)], srcv)
    pltpu.sync_copy(dst_h.at[pl.ds(base, E_PER_TILE)], dstv)

    def vec(j, carry):
        s16 = srcv[pl.ds(j * 16, 16)]
        d16 = dstv[pl.ds(j * 16, 16)]
        m = plsc.load_gather(keepv, [s16]) * plsc.load_gather(keepv, [d16])
        updv[pl.ds(j * 16, 16)] = m
        dstmv[pl.ds(j * 16, 16)] = jnp.where(
            m > 0.0, d16, N + (d16 & 127))
        return carry

    lax.fori_loop(0, E_PER_TILE // 16, vec, 0)
    plsc.subcore_barrier()
    pltpu.sync_copy(updv, deg_sh.at[dstmv], add=True)
    pltpu.sync_copy(dstmv, dstm_h.at[pl.ds(base, E_PER_TILE)])
    plsc.subcore_barrier()
    pltpu.sync_copy(deg_sh.at[pl.ds(r0, ROWS_PER_TILE)],
                    degp_h.at[c, pl.ds(r0, ROWS_PER_TILE)])


def _scalar_pass(src, dst, keep, zeros_n):
    fn = pl.kernel(
        _scalar_pass_body,
        out_type=(jax.ShapeDtypeStruct((2, N_PAD), jnp.float32),
                  jax.ShapeDtypeStruct((E,), jnp.int32)),
        mesh=_mesh(),
        compiler_params=pltpu.CompilerParams(needs_layout_passes=False),
        scratch_types=[
            pltpu.VMEM((N_PAD,), jnp.float32),
            pltpu.VMEM((E_PER_TILE,), jnp.int32),
            pltpu.VMEM((E_PER_TILE,), jnp.int32),
            pltpu.VMEM((E_PER_TILE,), jnp.float32),
            pltpu.VMEM((E_PER_TILE,), jnp.int32),
            pltpu.VMEM_SHARED((N_PAD,), jnp.float32),
        ],
    )
    return fn(src, dst, keep, zeros_n)


def _make_score_body(ncols):
    def body(hv_h, src_h, dstm_h, z_h, aggp_h, *refs):
        srcv, dstv = refs[0], refs[1]
        hv = refs[2:2 + ncols]
        updv = refs[2 + ncols:2 + 2 * ncols]
        accs = refs[2 + 2 * ncols:2 + 3 * ncols]
        c = lax.axis_index("c")
        s = lax.axis_index("s")
        wid = s * 2 + c
        r0 = s * ROWS_PER_TILE
        for t in range(ncols):
            pltpu.sync_copy(hv_h.at[pl.ds(t * N_PAD, N_PAD)], hv[t])
            pltpu.sync_copy(z_h.at[pl.ds(r0, ROWS_PER_TILE)],
                            accs[t].at[pl.ds(r0, ROWS_PER_TILE)])
        base = wid * E_PER_TILE
        pltpu.sync_copy(src_h.at[pl.ds(base, E_PER_TILE)], srcv)
        pltpu.sync_copy(dstm_h.at[pl.ds(base, E_PER_TILE)], dstv)

        def vec(j, carry):
            s16 = srcv[pl.ds(j * 16, 16)]
            for t in range(ncols):
                updv[t][pl.ds(j * 16, 16)] = plsc.load_gather(hv[t], [s16])
            return carry

        lax.fori_loop(0, E_PER_TILE // 16, vec, 0)
        plsc.subcore_barrier()
        for t in range(ncols):
            pltpu.sync_copy(updv[t], accs[t].at[dstv], add=True)
        plsc.subcore_barrier()
        for t in range(ncols):
            pltpu.sync_copy(accs[t].at[pl.ds(r0, ROWS_PER_TILE)],
                            aggp_h.at[c, pl.ds(t * N_PAD + r0, ROWS_PER_TILE)])
    return body


def _score_pass(hflat, src, dstm, zeros_n, ncols):
    fn = pl.kernel(
        _make_score_body(ncols),
        out_type=jax.ShapeDtypeStruct((2, ncols * N_PAD), jnp.float32),
        mesh=_mesh(),
        compiler_params=pltpu.CompilerParams(needs_layout_passes=False),
        scratch_types=(
            [pltpu.VMEM((E_PER_TILE,), jnp.int32)] * 2
            + [pltpu.VMEM((N_PAD,), jnp.float32)] * ncols
            + [pltpu.VMEM((E_PER_TILE,), jnp.float32)] * ncols
            + [pltpu.VMEM_SHARED((N_PAD,), jnp.float32)] * ncols
        ),
    )
    return fn(hflat, src, dstm, zeros_n)


EB_F = 160


def _edge_pass_body(h4_h, src4_h, dstm_h, z_h, agg4_h,
                    srcsup, dstb, rows0, rows1, feat_sh, sem0, sem1):
    c = lax.axis_index("c")
    s = lax.axis_index("s")
    r0 = s * ROWS_PER_TILE
    ept = E // 16
    nblk = ept // EB_F
    for cc in range(2):
        ch = c + 2 * cc
        pltpu.sync_copy(z_h.at[pl.ds(r0, ROWS_PER_TILE)],
                        feat_sh.at[pl.ds(r0, ROWS_PER_TILE)])
        plsc.subcore_barrier()
        base2 = s * ept

        SUP = 25

        def supload(w):
            pltpu.sync_copy(
                src4_h.at[pl.ds(ch * E + base2 + w * SUP * EB_F, SUP * EB_F)],
                srcsup)

        def gstart(i, rows, sem):
            pltpu.make_async_copy(
                h4_h.at[srcsup.at[pl.ds((i % SUP) * EB_F, EB_F)]],
                rows, sem).start()

        def gwait(rows, sem):
            pltpu.make_async_copy(h4_h.at[srcsup.at[pl.ds(0, EB_F)]],
                                  rows, sem).wait()

        def scat(i, rows):
            pltpu.sync_copy(dstm_h.at[pl.ds(base2 + i * EB_F, EB_F)], dstb)
            pltpu.sync_copy(rows, feat_sh.at[dstb], add=True)

        supload(0)
        gstart(0, rows0, sem0)

        def pair(p, carry):
            i0 = 2 * p
            gwait(rows0, sem0)

            @pl.when((i0 + 1) % SUP == 0)
            def _():
                supload((i0 + 1) // SUP)

            gstart(i0 + 1, rows1, sem1)
            scat(i0, rows0)
            gwait(rows1, sem1)

            @pl.when((i0 + 2) % SUP == 0)
            def _():
                supload((i0 + 2) // SUP)

            gstart(i0 + 2, rows0, sem0)
            scat(i0 + 1, rows1)
            return carry

        lax.fori_loop(0, (nblk - 1) // 2, pair, 0)
        gwait(rows0, sem0)
        scat(nblk - 1, rows0)
        plsc.subcore_barrier()
        pltpu.sync_copy(feat_sh.at[pl.ds(r0, ROWS_PER_TILE)],
                        agg4_h.at[pl.ds(ch * N_PAD + r0, ROWS_PER_TILE)])
        plsc.subcore_barrier()


def _edge_pass(h4flat, src4, dstm, zeros_l):
    fn = pl.kernel(
        _edge_pass_body,
        out_type=jax.ShapeDtypeStruct((4 * N_PAD, 128), jnp.float32),
        mesh=_mesh(),
        compiler_params=pltpu.CompilerParams(needs_layout_passes=False),
        scratch_types=[
            pltpu.VMEM((25 * EB_F,), jnp.int32),
            pltpu.VMEM((EB_F,), jnp.int32),
            pltpu.VMEM((EB_F, 128), jnp.float32),
            pltpu.VMEM((EB_F, 128), jnp.float32),
            pltpu.VMEM_SHARED((N_PAD, 128), jnp.float32),
            pltpu.SemaphoreType.DMA,
            pltpu.SemaphoreType.DMA,
        ],
    )
    return fn(h4flat, src4, dstm, zeros_l)


BN = 256
NB = N_PAD // BN


def _elu(x):
    return jnp.where(x > 0, x, jnp.exp(jnp.minimum(x, 0.0)) - 1.0)


def _prologue_x(out_prev, score_prev, keep_prev):
    t = jnp.tanh(score_prev)
    return _elu(out_prev * t[:, None]) * keep_prev[:, None]


def _matmul_body(has_skip, has_score, raw_x,
                 x_ref, s_ref, k_ref, w_ref, b_ref, d0_ref, d1_ref,
                 h4_ref, hs_ref, skip_ref):
    if raw_x:
        xb = x_ref[...]
    else:
        xb = _prologue_x(x_ref[...], s_ref[...], k_ref[...])
    rsq = lax.rsqrt(d0_ref[...] + d1_ref[...] + 1.0)
    hall = jnp.dot(xb, w_ref[...], preferred_element_type=jnp.float32) + b_ref[...]
    h4 = (hall[:, :512] * rsq[:, None]).reshape(BN, 4, 128)
    h4_ref[...] = jnp.transpose(h4, (1, 0, 2))
    if has_score:
        hs_ref[...] = hall[:, 512] * rsq
    if has_skip:
        skip_ref[...] = hall[:, 528:784]


def _matmul_layer(x_in, score_in, keep_in, wfull, bfull, d0, d1,
                  raw_x, has_skip, kdim, xwidth):
    outs = [jax.ShapeDtypeStruct((4, N_PAD, 128), jnp.float32),
            jax.ShapeDtypeStruct((N_PAD,), jnp.float32)]
    out_specs = [pl.BlockSpec((4, BN, 128), lambda i: (0, i, 0)),
                 pl.BlockSpec((BN,), lambda i: (i,))]
    if has_skip:
        outs.append(jax.ShapeDtypeStruct((N_PAD, 256), jnp.float32))
        out_specs.append(pl.BlockSpec((BN, 256), lambda i: (i, 0)))

    def wrapped(x_ref, s_ref, k_ref, w_ref, b_ref, d0_ref, d1_ref, *orefs):
        skip_ref = orefs[2] if has_skip else None
        _matmul_body(has_skip, True, raw_x, x_ref, s_ref, k_ref, w_ref,
                     b_ref, d0_ref, d1_ref, orefs[0], orefs[1], skip_ref)

    wwidth = wfull.shape[1]
    in_specs = [
        pl.BlockSpec((BN, xwidth), lambda i: (i, 0)),
        pl.BlockSpec((BN,), lambda i: (i,)),
        pl.BlockSpec((BN,), lambda i: (i,)),
        pl.BlockSpec((kdim, wwidth), lambda i: (0, 0)),
        pl.BlockSpec((wwidth,), lambda i: (0,)),
        pl.BlockSpec((BN,), lambda i: (i,)),
        pl.BlockSpec((BN,), lambda i: (i,)),
    ]
    return pl.pallas_call(
        wrapped, grid=(NB,), in_specs=in_specs,
        out_specs=out_specs, out_shape=outs,
    )(x_in, score_in, keep_in, wfull, bfull, d0, d1)


def _combine_body(has_skip, agg_ref, h4_ref, s0_ref, s1_ref, hs_ref,
                  d0_ref, d1_ref, valid_ref, skip_ref, out_ref, score_ref):
    rsq = lax.rsqrt(d0_ref[...] + d1_ref[...] + 1.0)
    gcn = rsq[None, :, None] * (agg_ref[...] + h4_ref[...])
    m = gcn[0:2]
    g = jax.nn.sigmoid(gcn[2:4])
    out = jnp.transpose(g * m, (1, 0, 2)).reshape(BN, 256)
    if has_skip:
        out = out + skip_ref[...]
    out_ref[...] = out
    sc = (s0_ref[...] + s1_ref[...] + hs_ref[...]) * rsq
    score_ref[...] = jnp.where(valid_ref[...] > 0, sc, -jnp.inf)


def _combine(agg4, h4, aggsp, hs, d0, d1, valid, skip):
    has_skip = skip is not None

    def wrapped(*refs):
        if has_skip:
            (agg_ref, h4_ref, s0r, s1r, hs_ref, d0r, d1r, vr, skr,
             out_ref, score_ref) = refs
        else:
            (agg_ref, h4_ref, s0r, s1r, hs_ref, d0r, d1r, vr,
             out_ref, score_ref) = refs
            skr = None
        _combine_body(has_skip, agg_ref, h4_ref, s0r, s1r, hs_ref,
                      d0r, d1r, vr, skr, out_ref, score_ref)

    in_specs = [
        pl.BlockSpec((4, BN, 128), lambda i: (0, i, 0)),
        pl.BlockSpec((4, BN, 128), lambda i: (0, i, 0)),
        pl.BlockSpec((BN,), lambda i: (i,)),
        pl.BlockSpec((BN,), lambda i: (i,)),
        pl.BlockSpec((BN,), lambda i: (i,)),
        pl.BlockSpec((BN,), lambda i: (i,)),
        pl.BlockSpec((BN,), lambda i: (i,)),
        pl.BlockSpec((BN,), lambda i: (i,)),
    ]
    args = [agg4, h4, aggsp[0], aggsp[1], hs, d0, d1, valid]
    if has_skip:
        in_specs.append(pl.BlockSpec((BN, 256), lambda i: (i, 0)))
        args.append(skip)
    return pl.pallas_call(
        wrapped, grid=(NB,), in_specs=in_specs,
        out_specs=[pl.BlockSpec((BN, 256), lambda i: (i, 0)),
                   pl.BlockSpec((BN,), lambda i: (i,))],
        out_shape=[jax.ShapeDtypeStruct((N_PAD, 256), jnp.float32),
                   jax.ShapeDtypeStruct((N_PAD,), jnp.float32)],
    )(*args)


def _topk_body(kk, score_ref, keep_ref):
    v = lax.bitcast_convert_type(score_ref[...], jnp.int32)
    k = jnp.where(v < 0, ~v, v ^ MININT) ^ MININT

    def step(b, res):
        cand = res | (jnp.int32(1) << (31 - b))
        cnt = jnp.sum((k >= (cand ^ MININT)).astype(jnp.int32))
        return jnp.where(cnt >= kk, cand, res)

    res = lax.fori_loop(0, 32, step, jnp.int32(0))
    keep_ref[...] = (k >= (res ^ MININT)).astype(jnp.float32)


def _topk_mask(score, kk):
    return pl.pallas_call(
        functools.partial(_topk_body, kk),
        out_shape=jax.ShapeDtypeStruct((N_PAD,), jnp.float32),
    )(score)


def _final_body(aggp_ref, h3_ref, d0_ref, d1_ref, keep_ref,
                w1_ref, b1_ref, w2_ref, b2_ref, out_ref):
    rsq = lax.rsqrt(d0_ref[...] + d1_ref[...] + 1.0)

    def _col(t):
        sl = pl.ds(t * N_PAD, N_PAD)
        return rsq * (aggp_ref[0, sl] + aggp_ref[1, sl] + h3_ref[sl])

    x3 = 0.5 * (jax.nn.sigmoid(_col(2)) * _col(0)
                + jax.nn.sigmoid(_col(3)) * _col(1))
    s = jnp.where(keep_ref[...] > 0, x3, -jnp.inf).reshape(80, 128)
    ri = (lax.broadcasted_iota(jnp.int32, (80, 128), 0) * 128
          + lax.broadcasted_iota(jnp.int32, (80, 128), 1))
    i32v = lax.broadcasted_iota(jnp.int32, (1, 32), 1)

    def step(i, carry):
        sv, pooled = carry
        mx = jnp.max(sv)
        jj = jnp.min(jnp.where(sv == mx, ri, jnp.int32(2 ** 30)))
        pooled = jnp.where(i32v == i, mx, pooled)
        sv = jnp.where(ri == jj, -jnp.inf, sv)
        return sv, pooled

    _, pooled = lax.fori_loop(0, 30, step,
                              (s, jnp.zeros((1, 32), jnp.float32)))
    t1 = _elu(jnp.dot(pooled, w1_ref[...],
                      preferred_element_type=jnp.float32) + b1_ref[...])
    t2 = jnp.dot(t1, w2_ref[...],
                 preferred_element_type=jnp.float32) + b2_ref[...]
    out_ref[...] = t2


def _final(aggp, h3, d0, d1, keep, w1, b1, w2, b2):
    return pl.pallas_call(
        _final_body,
        out_shape=jax.ShapeDtypeStruct((1, 10), jnp.float32),
    )(aggp, h3, d0, d1, keep, w1, b1, w2, b2)


def _matmul3_body(x_ref, s_ref, k_ref, w_ref, b_ref, d0_ref, d1_ref, *h3_ref):
    xb = _prologue_x(x_ref[...], s_ref[...], k_ref[...])
    rsq = lax.rsqrt(d0_ref[...] + d1_ref[...] + 1.0)
    hall = jnp.dot(xb, w_ref[...], preferred_element_type=jnp.float32) + b_ref[...]
    hs = hall * rsq[:, None]
    for t in range(4):
        h3_ref[t][...] = hs[:, t]


def _matmul3(out2, score2, keep2, w3cat, b3cat, d0, d1):
    return pl.pallas_call(
        _matmul3_body, grid=(NB,),
        in_specs=[
            pl.BlockSpec((BN, 256), lambda i: (i, 0)),
            pl.BlockSpec((BN,), lambda i: (i,)),
            pl.BlockSpec((BN,), lambda i: (i,)),
            pl.BlockSpec((256, 16), lambda i: (0, 0)),
            pl.BlockSpec((16,), lambda i: (0,)),
            pl.BlockSpec((BN,), lambda i: (i,)),
            pl.BlockSpec((BN,), lambda i: (i,)),
        ],
        out_specs=[pl.BlockSpec((BN,), lambda i: (i,))] * 4,
        out_shape=[jax.ShapeDtypeStruct((N_PAD,), jnp.float32)] * 4,
    )(out2, score2, keep2, w3cat, b3cat, d0, d1)


def _concat_weights(p, pre, has_skip):
    ws = [p[f'{pre}_W0'], p[f'{pre}_W1'], p[f'{pre}_Wg0'], p[f'{pre}_Wg1']]
    bs = [p[f'{pre}_b0'], p[f'{pre}_b1'], p[f'{pre}_bg0'], p[f'{pre}_bg1']]
    cols = [jnp.concatenate(ws, axis=1),
            jnp.pad(p[f'{pre}_Ws'], ((0, 0), (0, 15)))]
    bcols = [jnp.concatenate(bs), jnp.pad(p[f'{pre}_bs'], (0, 15))]
    if has_skip:
        cols.append(p[f'{pre}_Wskip'])
        bcols.append(jnp.zeros((256,), jnp.float32))
    return jnp.concatenate(cols, axis=1), jnp.concatenate(bcols)


def kernel(x, edge_index, params):
    p = params
    src = edge_index[0].astype(jnp.int32)
    dst = edge_index[1].astype(jnp.int32)
    xp = jnp.pad(x, ((0, N_PAD - N), (0, 0)))
    valid0 = (jnp.arange(N_PAD) < N).astype(jnp.float32)
    zeros_n = jnp.zeros((N_PAD,), jnp.float32)

    w1full, b1full = _concat_weights(p, 'c1', False)
    w2full, b2full = _concat_weights(p, 'c2', True)
    w3cat = jnp.pad(jnp.concatenate(
        [p['c3_W0'], p['c3_W1'], p['c3_Wg0'], p['c3_Wg1']], axis=1),
        ((0, 0), (0, 12)))
    b3cat = jnp.pad(jnp.concatenate(
        [p['c3_b0'], p['c3_b1'], p['c3_bg0'], p['c3_bg1']]), (0, 12))

    zeros_l = jnp.zeros((N_PAD, 128), jnp.float32)
    src4 = jnp.concatenate([src + ch * N_PAD for ch in range(4)])

    degp1, dstm1 = _scalar_pass(src, dst, valid0, zeros_n)
    d0, d1 = degp1[0], degp1[1]
    h4_1, hs_1 = _matmul_layer(xp, zeros_n, zeros_n, w1full, b1full,
                               d0, d1, True, False, F, F)
    agg4_1 = _edge_pass(h4_1.reshape(4 * N_PAD, 128), src4, dstm1, zeros_l)
    aggsp_1 = _score_pass(hs_1, src, dstm1, zeros_n, 1)
    out1, score1 = _combine(agg4_1.reshape(4, N_PAD, 128), h4_1, aggsp_1,
                            hs_1, d0, d1, valid0, None)
    keep1 = _topk_mask(score1, int(math.ceil(0.5 * N)))

    degp2, dstm2 = _scalar_pass(src, dst, keep1, zeros_n)
    d0, d1 = degp2[0], degp2[1]
    h4_2, hs_2, skip2 = _matmul_layer(out1, score1, keep1, w2full, b2full,
                                      d0, d1, False, True, 256, 256)
    agg4_2 = _edge_pass(h4_2.reshape(4 * N_PAD, 128), src4, dstm2, zeros_l)
    aggsp_2 = _score_pass(hs_2, src, dstm2, zeros_n, 1)
    out2, score2 = _combine(agg4_2.reshape(4, N_PAD, 128), h4_2, aggsp_2,
                            hs_2, d0, d1, keep1, skip2)
    keep2 = _topk_mask(score2, int(math.ceil(0.5 * math.ceil(0.5 * N))))

    degp3, dstm3 = _scalar_pass(src, dst, keep2, zeros_n)
    d0, d1 = degp3[0], degp3[1]
    h3cols = _matmul3(out2, score2, keep2, w3cat, b3cat, d0, d1)
    h3flat = jnp.concatenate(h3cols)
    aggp3 = _score_pass(h3flat, src, dstm3, zeros_n, 4)
    w1p = jnp.pad(p['cls_W1'], ((0, 2), (0, 0)))
    return _final(aggp3, h3flat, d0, d1, keep2,
                  w1p, p['cls_b1'].reshape(1, -1),
                  p['cls_W2'], p['cls_b2'].reshape(1, -1))
